# Initial kernel scaffold; baseline (speedup 1.0000x reference)
import jax
import jax.numpy as jnp
from jax.experimental import pallas as pl


def kernel(x, pe):
    raise NotImplementedError("write your pallas kernel here")


# speedup vs baseline: 2.1912x; 2.1912x over previous
import functools

import jax
import jax.numpy as jnp
from jax import lax
from jax.experimental import pallas as pl
from jax.experimental.pallas import tpu as pltpu
from jax.experimental.pallas import tpu_sc as plsc

_D = 1024
_N = 4 * 8192
_NW = 32
_PER_W = _N // _NW
_CHUNK = 64
_NCHUNK = _PER_W // _CHUNK

_mesh = plsc.VectorSubcoreMesh(core_axis_name="c", subcore_axis_name="s")


@functools.partial(
    pl.kernel,
    mesh=_mesh,
    out_type=jax.ShapeDtypeStruct((_N, _D), jnp.float32),
    scratch_types=[
        pltpu.VMEM((_NCHUNK, _CHUNK), jnp.int32),
        pltpu.VMEM((_CHUNK, _D), jnp.float32),
        pltpu.SemaphoreType.DMA,
    ],
)
def _gather(x_hbm, pe_hbm, out_hbm, idx_v, rows_v, sem):
    wid = lax.axis_index("s") * 2 + lax.axis_index("c")
    base = wid * _PER_W
    pltpu.sync_copy(x_hbm.at[wid], idx_v)

    def body(i, carry):
        pltpu.async_copy(pe_hbm.at[idx_v.at[i]], rows_v, sem).wait()
        pltpu.sync_copy(rows_v, out_hbm.at[pl.ds(base + i * _CHUNK, _CHUNK)])
        return carry

    lax.fori_loop(0, _NCHUNK, body, 0)


def kernel(x, pe):
    xr = x.reshape(_NW, _NCHUNK, _CHUNK)
    out = _gather(xr, pe)
    return out.reshape(x.shape[0], x.shape[1], _D)


# speedup vs baseline: 2.3716x; 1.0823x over previous
import functools

import jax
import jax.numpy as jnp
from jax import lax
from jax.experimental import pallas as pl
from jax.experimental.pallas import tpu as pltpu
from jax.experimental.pallas import tpu_sc as plsc

_D = 1024
_N = 4 * 8192
_NW = 32
_PER_W = _N // _NW
_CHUNK = 32
_NCHUNK = _PER_W // _CHUNK
_NBUF = 2

_mesh = plsc.VectorSubcoreMesh(core_axis_name="c", subcore_axis_name="s")


@functools.partial(
    pl.kernel,
    mesh=_mesh,
    out_type=jax.ShapeDtypeStruct((_N, _D), jnp.float32),
    scratch_types=[
        pltpu.VMEM((_NCHUNK, _CHUNK), jnp.int32),
        pltpu.VMEM((_NBUF, _CHUNK, _D), jnp.float32),
        pltpu.SemaphoreType.DMA,
        pltpu.SemaphoreType.DMA,
        pltpu.SemaphoreType.DMA,
        pltpu.SemaphoreType.DMA,
    ],
)
def _gather(x_hbm, pe_hbm, out_hbm, idx_v, rows_v, g0, g1, s0, s1):
    wid = lax.axis_index("s") * 2 + lax.axis_index("c")
    base = wid * _PER_W
    pltpu.sync_copy(x_hbm.at[wid], idx_v)
    gsems = (g0, g1)
    ssems = (s0, s1)

    for b in range(_NBUF):
        pltpu.async_copy(pe_hbm.at[idx_v.at[b]], rows_v.at[b], gsems[b])

    def body(og, carry):
        for b in range(_NBUF):
            c = og * _NBUF + b
            pltpu.make_async_copy(
                pe_hbm.at[idx_v.at[c]], rows_v.at[b], gsems[b]).wait()
            st = pltpu.async_copy(
                rows_v.at[b],
                out_hbm.at[pl.ds(base + c * _CHUNK, _CHUNK)],
                ssems[b])
            st.wait()

            @pl.when(c + _NBUF < _NCHUNK)
            def _():
                pltpu.async_copy(
                    pe_hbm.at[idx_v.at[c + _NBUF]], rows_v.at[b], gsems[b])

        return carry

    lax.fori_loop(0, _NCHUNK // _NBUF, body, 0)


def kernel(x, pe):
    xr = x.reshape(_NW, _NCHUNK, _CHUNK)
    out = _gather(xr, pe)
    return out.reshape(x.shape[0], x.shape[1], _D)
